# baseline (speedup 1.0000x reference)
import functools

import jax
import jax.numpy as jnp
from jax import lax
from jax.experimental import pallas as pl
from jax.experimental.pallas import tpu as pltpu
from jax.experimental.pallas import tpu_sc as plsc

NUM_CORES = 2
NUM_SUBCORES = 16
NW = NUM_CORES * NUM_SUBCORES
IDX_CHUNK = 128


@functools.partial(jax.jit, static_argnums=(2, 3))
def _embed_gather(table, idx3d, b_per_w, n_chunks):
    D = table.shape[1]
    B = NW * b_per_w
    mesh = plsc.VectorSubcoreMesh(core_axis_name="c", subcore_axis_name="s")

    @functools.partial(
        pl.kernel,
        mesh=mesh,
        out_type=jax.ShapeDtypeStruct((B, D), jnp.float32),
        scratch_types=[
            pltpu.VMEM((n_chunks, IDX_CHUNK), jnp.int32),
            pltpu.VMEM((b_per_w, D), jnp.float32),
            pltpu.SemaphoreType.DMA,
        ],
        compiler_params=pltpu.CompilerParams(use_tc_tiling_on_sc=False),
    )
    def k(table_hbm, idx_hbm, out_hbm, idx_v, rows_v, sem):
        wid = lax.axis_index("s") * NUM_CORES + lax.axis_index("c")
        base = wid * b_per_w
        pltpu.sync_copy(idx_hbm.at[wid], idx_v)
        copies = []
        for j in range(n_chunks):
            copies.append(
                pltpu.async_copy(
                    table_hbm.at[idx_v.at[j]],
                    rows_v.at[pl.ds(j * IDX_CHUNK, IDX_CHUNK)],
                    sem,
                )
            )
        for c in copies:
            c.wait()
        pltpu.sync_copy(rows_v, out_hbm.at[pl.ds(base, b_per_w)])

    return k(table, idx3d)


def kernel(labels, table):
    B = labels.shape[0]
    b_per_w = B // NW
    n_chunks = b_per_w // IDX_CHUNK
    idx3d = labels.astype(jnp.int32).reshape(NW, n_chunks, IDX_CHUNK)
    return _embed_gather(table, idx3d, b_per_w, n_chunks)


# speedup vs baseline: 6.5610x; 6.5610x over previous
import functools

import jax
import jax.numpy as jnp
from jax import lax
from jax.experimental import pallas as pl
from jax.experimental.pallas import tpu as pltpu
from jax.experimental.pallas import tpu_sc as plsc

NUM_CORES = 2
NUM_SUBCORES = 16
NW = NUM_CORES * NUM_SUBCORES
CH = 128
G = 16


@functools.partial(jax.jit, static_argnums=(2,))
def _embed_gather_t(table3, labels, b_per_w):
    B = NW * b_per_w
    n_ch = b_per_w // CH
    mesh = plsc.VectorSubcoreMesh(core_axis_name="c", subcore_axis_name="s")

    @functools.partial(
        pl.kernel,
        mesh=mesh,
        out_type=jax.ShapeDtypeStruct((4, 8, B), jnp.float32),
        scratch_types=[
            pltpu.VMEM((b_per_w,), jnp.int32),
            pltpu.SMEM((b_per_w,), jnp.int32),
            pltpu.VMEM_SHARED((NUM_SUBCORES, 2 * NUM_CORES * b_per_w), jnp.int32),
            pltpu.VMEM((4, 8, CH * 8), jnp.float32),
            pltpu.VMEM((4, 8, b_per_w), jnp.float32),
            pltpu.SemaphoreType.DMA,
            pltpu.SemaphoreType.DMA,
        ],
        compiler_params=pltpu.CompilerParams(needs_layout_passes=False),
    )
    def k(table_hbm, lab_hbm, out_hbm, idx_v, idx_s, idx_sh, grp_v, rows_v,
          sem, lsem):
        cid = lax.axis_index("c")
        sid = lax.axis_index("s")
        wid = sid * NUM_CORES + cid
        base = wid * b_per_w
        sh = idx_sh.at[sid, pl.ds(cid * b_per_w, b_per_w)]
        cp1 = pltpu.async_copy(lab_hbm.at[pl.ds(base, b_per_w)], idx_v, lsem)
        cp2 = pltpu.async_copy(lab_hbm.at[pl.ds(base, b_per_w)], sh, lsem)
        cp1.wait()
        cp2.wait()
        pltpu.async_copy(sh, idx_s, lsem).wait()

        lane16 = lax.iota(jnp.int32, 16)

        for c in range(n_ch):
            def fetch_body(g, carry, c=c):
                copies = []
                for j in range(G):
                    i = g * G + j
                    l8 = (idx_s[c * CH + i] // 8) * 8
                    copies.append(
                        pltpu.async_copy(
                            table_hbm.at[:, :, pl.ds(l8, 8)],
                            grp_v.at[:, :, pl.ds(i * 8, 8)],
                            sem,
                        )
                    )
                for cp in copies:
                    cp.wait()
                return carry

            lax.fori_loop(0, CH // G, fetch_body, 0)

            def sel_body(g, carry, c=c):
                j0 = g * 16
                lv = idx_v[pl.ds(c * CH + j0, 16)]
                pos = (lane16 + j0) * 8 + lax.rem(lv, 8)
                for d in range(32):
                    tr, r = d // 8, d % 8
                    vals = plsc.load_gather(
                        grp_v,
                        [
                            jnp.full((16,), tr, dtype=jnp.int32),
                            jnp.full((16,), r, dtype=jnp.int32),
                            pos,
                        ],
                    )
                    rows_v[tr, r, pl.ds(c * CH + j0, 16)] = vals
                return carry

            lax.fori_loop(0, CH // G, sel_body, 0)

        pltpu.sync_copy(rows_v, out_hbm.at[:, :, pl.ds(base, b_per_w)])

    return k(table3, labels)


def kernel(labels, table):
    B = labels.shape[0]
    b_per_w = B // NW
    table3 = table.T.reshape(4, 8, table.shape[0])
    out3 = _embed_gather_t(table3, labels.astype(jnp.int32), b_per_w)
    return out3.reshape(32, B).T


# speedup vs baseline: 8.8443x; 1.3480x over previous
import functools

import jax
import jax.numpy as jnp
from jax import lax
from jax.experimental import pallas as pl
from jax.experimental.pallas import tpu as pltpu
from jax.experimental.pallas import tpu_sc as plsc

NUM_CORES = 2
NUM_SUBCORES = 16
NW = NUM_CORES * NUM_SUBCORES
CH = 128
G = 16


@functools.partial(jax.jit, static_argnums=(2,))
def _embed_gather_t(table3, labels, b_per_w):
    B = NW * b_per_w
    n_ch = b_per_w // CH
    mesh = plsc.VectorSubcoreMesh(core_axis_name="c", subcore_axis_name="s")

    @functools.partial(
        pl.kernel,
        mesh=mesh,
        out_type=jax.ShapeDtypeStruct((4, 8, B), jnp.float32),
        scratch_types=[
            pltpu.VMEM((b_per_w,), jnp.int32),
            pltpu.SMEM((b_per_w,), jnp.int32),
            pltpu.VMEM_SHARED((NUM_SUBCORES, 2 * NUM_CORES * b_per_w), jnp.int32),
            pltpu.VMEM((2, 4, 8, CH * 8), jnp.float32),
            pltpu.VMEM((4, 8, b_per_w), jnp.float32),
            pltpu.SemaphoreType.DMA,
            pltpu.SemaphoreType.DMA,
            pltpu.SemaphoreType.DMA,
        ],
        compiler_params=pltpu.CompilerParams(needs_layout_passes=False),
    )
    def k(table_hbm, lab_hbm, out_hbm, idx_v, idx_s, idx_sh, grp_v, rows_v,
          sem0, sem1, lsem):
        sems = (sem0, sem1)
        cid = lax.axis_index("c")
        sid = lax.axis_index("s")
        wid = sid * NUM_CORES + cid
        base = wid * b_per_w
        sh = idx_sh.at[sid, pl.ds(cid * b_per_w, b_per_w)]
        cp1 = pltpu.async_copy(lab_hbm.at[pl.ds(base, b_per_w)], idx_v, lsem)
        cp2 = pltpu.async_copy(lab_hbm.at[pl.ds(base, b_per_w)], sh, lsem)
        cp1.wait()
        cp2.wait()
        pltpu.async_copy(sh, idx_s, lsem).wait()

        lane16 = lax.iota(jnp.int32, 16)

        def issue_chunk(c, buf):
            def fetch_body(g, carry):
                for j in range(G):
                    i = g * G + j
                    l8 = (idx_s[c * CH + i] // 8) * 8
                    pltpu.async_copy(
                        table_hbm.at[:, :, pl.ds(l8, 8)],
                        grp_v.at[buf, :, :, pl.ds(i * 8, 8)],
                        sems[buf],
                    )
                return carry

            lax.fori_loop(0, CH // G, fetch_body, 0)

        def drain_chunk(buf):
            pltpu.make_async_copy(
                table_hbm.at[:, :, pl.ds(0, CH * 8)],
                grp_v.at[buf],
                sems[buf],
            ).wait()

        def select_chunk(c, buf):
            def sel_body(g, carry):
                j0 = g * 16
                lv = idx_v[pl.ds(c * CH + j0, 16)]
                pos = (lane16 + j0) * 8 + lax.rem(lv, 8)
                for d in range(32):
                    tr, r = d // 8, d % 8
                    vals = plsc.load_gather(
                        grp_v,
                        [
                            jnp.full((16,), buf, dtype=jnp.int32),
                            jnp.full((16,), tr, dtype=jnp.int32),
                            jnp.full((16,), r, dtype=jnp.int32),
                            pos,
                        ],
                    )
                    rows_v[tr, r, pl.ds(c * CH + j0, 16)] = vals
                return carry

            lax.fori_loop(0, CH // G, sel_body, 0)

        issue_chunk(0, 0)
        for c in range(1, n_ch):
            issue_chunk(c, c % 2)
            drain_chunk((c - 1) % 2)
            select_chunk(c - 1, (c - 1) % 2)
        drain_chunk((n_ch - 1) % 2)
        select_chunk(n_ch - 1, (n_ch - 1) % 2)

        pltpu.sync_copy(rows_v, out_hbm.at[:, :, pl.ds(base, b_per_w)])

    return k(table3, labels)


def kernel(labels, table):
    B = labels.shape[0]
    b_per_w = B // NW
    table3 = table.T.reshape(4, 8, table.shape[0])
    out3 = _embed_gather_t(table3, labels.astype(jnp.int32), b_per_w)
    return out3.reshape(32, B).T


# speedup vs baseline: 8.8539x; 1.0011x over previous
import functools

import jax
import jax.numpy as jnp
from jax import lax
from jax.experimental import pallas as pl
from jax.experimental.pallas import tpu as pltpu
from jax.experimental.pallas import tpu_sc as plsc

NUM_CORES = 2
NUM_SUBCORES = 16
NW = NUM_CORES * NUM_SUBCORES
CH = 128
G = 16


@functools.partial(jax.jit, static_argnums=(2,))
def _embed_gather_t(table3, labels, b_per_w):
    B = NW * b_per_w
    n_ch = b_per_w // CH
    mesh = plsc.VectorSubcoreMesh(core_axis_name="c", subcore_axis_name="s")

    @functools.partial(
        pl.kernel,
        mesh=mesh,
        out_type=jax.ShapeDtypeStruct((4, 8, B), jnp.float32),
        scratch_types=[
            pltpu.VMEM((b_per_w,), jnp.int32),
            pltpu.SMEM((b_per_w,), jnp.int32),
            pltpu.VMEM_SHARED((NUM_SUBCORES, 2 * NUM_CORES * b_per_w), jnp.int32),
            pltpu.VMEM((3, 4, 8, CH * 8), jnp.float32),
            pltpu.VMEM((4, 8, b_per_w), jnp.float32),
            pltpu.SemaphoreType.DMA,
            pltpu.SemaphoreType.DMA,
            pltpu.SemaphoreType.DMA,
            pltpu.SemaphoreType.DMA,
        ],
        compiler_params=pltpu.CompilerParams(needs_layout_passes=False),
    )
    def k(table_hbm, lab_hbm, out_hbm, idx_v, idx_s, idx_sh, grp_v, rows_v,
          sem0, sem1, sem2, lsem):
        sems = (sem0, sem1, sem2)
        cid = lax.axis_index("c")
        sid = lax.axis_index("s")
        wid = sid * NUM_CORES + cid
        base = wid * b_per_w
        sh = idx_sh.at[sid, pl.ds(cid * b_per_w, b_per_w)]
        cp1 = pltpu.async_copy(lab_hbm.at[pl.ds(base, b_per_w)], idx_v, lsem)
        cp2 = pltpu.async_copy(lab_hbm.at[pl.ds(base, b_per_w)], sh, lsem)
        cp1.wait()
        cp2.wait()
        pltpu.async_copy(sh, idx_s, lsem).wait()

        lane16 = lax.iota(jnp.int32, 16)

        def issue_chunk(c, buf):
            def fetch_body(g, carry):
                for j in range(G):
                    i = g * G + j
                    l8 = (idx_s[c * CH + i] // 8) * 8
                    pltpu.async_copy(
                        table_hbm.at[:, :, pl.ds(l8, 8)],
                        grp_v.at[buf, :, :, pl.ds(i * 8, 8)],
                        sems[buf],
                    )
                return carry

            lax.fori_loop(0, CH // G, fetch_body, 0)

        def drain_chunk(buf):
            pltpu.make_async_copy(
                table_hbm.at[:, :, pl.ds(0, CH * 8)],
                grp_v.at[buf],
                sems[buf],
            ).wait()

        def select_chunk(c, buf):
            def sel_body(g, carry):
                j0 = g * 16
                lv = idx_v[pl.ds(c * CH + j0, 16)]
                pos = (lane16 + j0) * 8 + lax.rem(lv, 8)
                for d in range(32):
                    tr, r = d // 8, d % 8
                    vals = plsc.load_gather(
                        grp_v,
                        [
                            jnp.full((16,), buf, dtype=jnp.int32),
                            jnp.full((16,), tr, dtype=jnp.int32),
                            jnp.full((16,), r, dtype=jnp.int32),
                            pos,
                        ],
                    )
                    rows_v[tr, r, pl.ds(c * CH + j0, 16)] = vals
                return carry

            lax.fori_loop(0, CH // G, sel_body, 0)

        issue_chunk(0, 0)
        issue_chunk(1, 1)
        for c in range(2, n_ch):
            issue_chunk(c, c % 3)
            drain_chunk((c - 2) % 3)
            select_chunk(c - 2, (c - 2) % 3)
        for c in range(n_ch - 2, n_ch):
            drain_chunk(c % 3)
            select_chunk(c, c % 3)

        pltpu.sync_copy(rows_v, out_hbm.at[:, :, pl.ds(base, b_per_w)])

    return k(table3, labels)


def kernel(labels, table):
    B = labels.shape[0]
    b_per_w = B // NW
    table3 = table.T.reshape(4, 8, table.shape[0])
    out3 = _embed_gather_t(table3, labels.astype(jnp.int32), b_per_w)
    return out3.reshape(32, B).T
